# initial kernel scaffold (unmeasured)
import jax
import jax.numpy as jnp
from jax import lax
from jax.experimental import pallas as pl
from jax.experimental.pallas import tpu as pltpu


def kernel(
    x,
):
    def body(*refs):
        pass

    out_shape = jax.ShapeDtypeStruct(..., jnp.float32)
    return pl.pallas_call(body, out_shape=out_shape)(...)


# baseline (device time: 10435 ns/iter reference)
import jax
import jax.numpy as jnp
from jax import lax
from jax.experimental import pallas as pl
from jax.experimental.pallas import tpu as pltpu

N_DEV = 16


def kernel(x):
    m_per, n = x.shape
    total_rows = N_DEV * m_per

    def body(x_ref, out_ref, local_ref, comm_ref, send_sems, recv_sems):
        me = lax.axis_index("i")

        barrier = pltpu.get_barrier_semaphore()
        for d in range(N_DEV):

            @pl.when(me != d)
            def _():
                pl.semaphore_signal(
                    barrier,
                    inc=1,
                    device_id=(d,),
                    device_id_type=pl.DeviceIdType.MESH,
                )

        pl.semaphore_wait(barrier, N_DEV - 1)

        local_ref[:, :] = jnp.sum(x_ref[:, :], axis=0, keepdims=True)

        for d in range(N_DEV):

            @pl.when(me != d)
            def _():
                rdma = pltpu.make_async_remote_copy(
                    src_ref=local_ref,
                    dst_ref=comm_ref.at[me],
                    send_sem=send_sems.at[d],
                    recv_sem=recv_sems.at[me],
                    device_id=(d,),
                    device_id_type=pl.DeviceIdType.MESH,
                )
                rdma.start()

        for p in range(N_DEV):

            @pl.when(me != p)
            def _():
                recv = pltpu.make_async_remote_copy(
                    src_ref=local_ref,
                    dst_ref=comm_ref.at[p],
                    send_sem=send_sems.at[p],
                    recv_sem=recv_sems.at[p],
                    device_id=(p,),
                    device_id_type=pl.DeviceIdType.MESH,
                )
                recv.wait_recv()

        acc = local_ref[:, :]
        for p in range(N_DEV):
            acc = acc + jnp.where(me == p, 0.0, comm_ref[p, :, :])
        out_ref[:, :] = acc * (1.0 / total_rows)

        for d in range(N_DEV):

            @pl.when(me != d)
            def _():
                send = pltpu.make_async_remote_copy(
                    src_ref=local_ref,
                    dst_ref=comm_ref.at[0],
                    send_sem=send_sems.at[d],
                    recv_sem=recv_sems.at[0],
                    device_id=(d,),
                    device_id_type=pl.DeviceIdType.MESH,
                )
                send.wait_send()

    return pl.pallas_call(
        body,
        out_shape=jax.ShapeDtypeStruct((1, n), jnp.float32),
        in_specs=[pl.BlockSpec(memory_space=pltpu.VMEM)],
        out_specs=pl.BlockSpec(memory_space=pltpu.VMEM),
        scratch_shapes=[
            pltpu.VMEM((1, n), jnp.float32),
            pltpu.VMEM((N_DEV, 1, n), jnp.float32),
            pltpu.SemaphoreType.DMA((N_DEV,)),
            pltpu.SemaphoreType.DMA((N_DEV,)),
        ],
        compiler_params=pltpu.CompilerParams(collective_id=0),
    )(x)


# device time: 10308 ns/iter; 1.0123x vs baseline; 1.0123x over previous
import jax
import jax.numpy as jnp
from jax import lax
from jax.experimental import pallas as pl
from jax.experimental.pallas import tpu as pltpu

N_DEV = 16


def kernel(x):
    m_per, n = x.shape
    total_rows = N_DEV * m_per

    def body(x_ref, out_ref, local_ref, comm_ref, send_sems, recv_sems):
        me = lax.axis_index("i")

        barrier = pltpu.get_barrier_semaphore()
        for d in range(N_DEV):

            @pl.when(me != d)
            def _():
                pl.semaphore_signal(
                    barrier,
                    inc=1,
                    device_id=(d,),
                    device_id_type=pl.DeviceIdType.MESH,
                )

        local_ref[:, :] = jnp.sum(x_ref[:, :], axis=0, keepdims=True)

        pl.semaphore_wait(barrier, N_DEV - 1)

        for d in range(N_DEV):

            @pl.when(me != d)
            def _():
                rdma = pltpu.make_async_remote_copy(
                    src_ref=local_ref,
                    dst_ref=comm_ref.at[me],
                    send_sem=send_sems.at[d],
                    recv_sem=recv_sems.at[me],
                    device_id=(d,),
                    device_id_type=pl.DeviceIdType.MESH,
                )
                rdma.start()

        for p in range(N_DEV):

            @pl.when(me != p)
            def _():
                recv = pltpu.make_async_remote_copy(
                    src_ref=local_ref,
                    dst_ref=comm_ref.at[p],
                    send_sem=send_sems.at[p],
                    recv_sem=recv_sems.at[p],
                    device_id=(p,),
                    device_id_type=pl.DeviceIdType.MESH,
                )
                recv.wait_recv()

        acc = local_ref[:, :]
        for p in range(N_DEV):
            acc = acc + jnp.where(me == p, 0.0, comm_ref[p, :, :])
        out_ref[:, :] = acc * (1.0 / total_rows)

        for d in range(N_DEV):

            @pl.when(me != d)
            def _():
                send = pltpu.make_async_remote_copy(
                    src_ref=local_ref,
                    dst_ref=comm_ref.at[0],
                    send_sem=send_sems.at[d],
                    recv_sem=recv_sems.at[0],
                    device_id=(d,),
                    device_id_type=pl.DeviceIdType.MESH,
                )
                send.wait_send()

    return pl.pallas_call(
        body,
        out_shape=jax.ShapeDtypeStruct((1, n), jnp.float32),
        in_specs=[pl.BlockSpec(memory_space=pltpu.VMEM)],
        out_specs=pl.BlockSpec(memory_space=pltpu.VMEM),
        scratch_shapes=[
            pltpu.VMEM((1, n), jnp.float32),
            pltpu.VMEM((N_DEV, 1, n), jnp.float32),
            pltpu.SemaphoreType.DMA((N_DEV,)),
            pltpu.SemaphoreType.DMA((N_DEV,)),
        ],
        compiler_params=pltpu.CompilerParams(collective_id=0),
    )(x)
